# baseline (speedup 1.0000x reference)
import functools

import jax
import jax.numpy as jnp
from jax import lax
from jax.experimental import pallas as pl
from jax.experimental.pallas import tpu as pltpu
from jax.experimental.pallas import tpu_sc as plsc

BATCH = 16384
SPEAKER_DIM = 32
SENT_DIM = 768
OUT_DIM = SENT_DIM + SPEAKER_DIM
N_SPK = 1000000
PACK = 128 // SPEAKER_DIM
SPK_BLK = 1024
ROWS_BLK = SPK_BLK // PACK
N_BLKS = (N_SPK + SPK_BLK - 1) // SPK_BLK
N_PACKED = N_BLKS * ROWS_BLK

NC = 2
NS = 16
NW = NC * NS
B_PER_W = BATCH // NW
CHUNK = 128
N_CHUNKS = B_PER_W // CHUNK
LANES = 16


def _tc_repack(table_t):
    def body(t_ref, o_ref):
        tt = jnp.swapaxes(t_ref[...], 0, 1)
        for p in range(PACK):
            o_ref[:, p * SPEAKER_DIM:(p + 1) * SPEAKER_DIM] = (
                tt[p * ROWS_BLK:(p + 1) * ROWS_BLK, :]
            )

    return pl.pallas_call(
        body,
        grid=(N_BLKS,),
        in_specs=[pl.BlockSpec((32, SPK_BLK), lambda i: (0, i))],
        out_specs=pl.BlockSpec((ROWS_BLK, 128), lambda i: (i, 0)),
        out_shape=jax.ShapeDtypeStruct((N_PACKED, 128), jnp.float32),
    )(table_t)


def _sc_gather(table4, idx3):
    mesh = plsc.VectorSubcoreMesh(core_axis_name="c", subcore_axis_name="s")

    @functools.partial(
        pl.kernel,
        mesh=mesh,
        out_type=jax.ShapeDtypeStruct((BATCH, 128), jnp.float32),
        scratch_types=[
            pltpu.VMEM((N_CHUNKS, CHUNK), jnp.int32),
            pltpu.VMEM((N_CHUNKS, CHUNK), jnp.int32),
            pltpu.VMEM((2, CHUNK, 128), jnp.float32),
            pltpu.SemaphoreType.DMA,
        ],
        compiler_params=pltpu.CompilerParams(use_tc_tiling_on_sc=True),
    )
    def gather_k(table_hbm, idx_hbm, out_hbm, idx_v, idx4_v, big_v, sem):
        wid = lax.axis_index("s") * NC + lax.axis_index("c")
        base = wid * B_PER_W
        pltpu.sync_copy(idx_hbm.at[wid], idx_v)

        def shift(k, _):
            for j in range(N_CHUNKS):
                iv = idx_v[j, pl.ds(k * LANES, LANES)]
                idx4_v[j, pl.ds(k * LANES, LANES)] = (
                    (iv >> 10) * ROWS_BLK + (iv & (ROWS_BLK - 1))
                )
            return ()

        lax.fori_loop(0, CHUNK // LANES, shift, ())

        def fire(j):
            return pltpu.async_copy(
                table_hbm.at[idx4_v.at[j]], big_v.at[j % 2], sem
            )

        pending = fire(0)
        for j in range(N_CHUNKS):
            pending.wait()
            if j + 1 < N_CHUNKS:
                pending = fire(j + 1)
            pltpu.sync_copy(
                big_v.at[j % 2], out_hbm.at[pl.ds(base + j * CHUNK, CHUNK)]
            )

    return gather_k(table4, idx3)


def _tc_sentence(sentence_embeddings):
    bm = 512
    grid = BATCH // bm

    def body(s_ref, o_ref):
        o_ref[...] = s_ref[...]

    return pl.pallas_call(
        body,
        grid=(grid,),
        in_specs=[pl.BlockSpec((bm, SENT_DIM), lambda i: (i, 0))],
        out_specs=pl.BlockSpec((bm, SENT_DIM), lambda i: (i, 0)),
        out_shape=jax.ShapeDtypeStruct((BATCH, OUT_DIM), jnp.float32),
    )(sentence_embeddings)


def _tc_insert(buf, packed, ids_col):
    bm = 2048
    grid = BATCH // bm

    def body(b_ref, g_ref, i_ref, o_ref):
        sel = (i_ref[...] >> 8) & (PACK - 1)
        acc = jnp.zeros((bm, SPEAKER_DIM), jnp.float32)
        for g in range(PACK):
            cand = g_ref[:, g * SPEAKER_DIM:(g + 1) * SPEAKER_DIM]
            acc = jnp.where(sel == g, cand, acc)
        o_ref[:, :SPEAKER_DIM] = acc

    return pl.pallas_call(
        body,
        grid=(grid,),
        in_specs=[
            pl.BlockSpec(memory_space=pltpu.MemorySpace.HBM),
            pl.BlockSpec((bm, 128), lambda i: (i, 0)),
            pl.BlockSpec((bm, 1), lambda i: (i, 0)),
        ],
        out_specs=pl.BlockSpec((bm, 128), lambda i: (i, SENT_DIM // 128)),
        out_shape=jax.ShapeDtypeStruct((BATCH, OUT_DIM), jnp.float32),
        input_output_aliases={0: 0},
    )(buf, packed, ids_col)


def kernel(sentence_embeddings, speaker_ids, speaker_table):
    ids = speaker_ids.astype(jnp.int32)
    idx3 = ids.reshape(NW, N_CHUNKS, CHUNK)
    table4 = _tc_repack(speaker_table.T)
    packed = _sc_gather(table4, idx3)
    buf = _tc_sentence(sentence_embeddings)
    return _tc_insert(buf, packed, ids.reshape(BATCH, 1))


# speedup vs baseline: 1.2860x; 1.2860x over previous
import functools

import jax
import jax.numpy as jnp
from jax import lax
from jax.experimental import pallas as pl
from jax.experimental.pallas import tpu as pltpu
from jax.experimental.pallas import tpu_sc as plsc

BATCH = 16384
SPEAKER_DIM = 32
SENT_DIM = 768
OUT_DIM = SENT_DIM + SPEAKER_DIM

NC = 2
NS = 16
NW = NC * NS
B_PER_W = BATCH // NW
CHUNK = 128
N_CHUNKS = B_PER_W // CHUNK


def _sc_gather(speaker_table, idx3):
    mesh = plsc.VectorSubcoreMesh(core_axis_name="c", subcore_axis_name="s")

    @functools.partial(
        pl.kernel,
        mesh=mesh,
        out_type=jax.ShapeDtypeStruct((BATCH, SPEAKER_DIM), jnp.float32),
        scratch_types=[
            pltpu.VMEM((N_CHUNKS, CHUNK), jnp.int32),
            pltpu.VMEM((N_CHUNKS, CHUNK, SPEAKER_DIM), jnp.float32),
            pltpu.SemaphoreType.DMA,
        ],
        compiler_params=pltpu.CompilerParams(use_tc_tiling_on_sc=False),
    )
    def gather_k(table_hbm, idx_hbm, out_hbm, idx_v, rows_v, sem):
        wid = lax.axis_index("s") * NC + lax.axis_index("c")
        pltpu.sync_copy(idx_hbm.at[wid], idx_v)
        copies = [
            pltpu.async_copy(table_hbm.at[idx_v.at[j]], rows_v.at[j], sem)
            for j in range(N_CHUNKS)
        ]
        for c in copies:
            c.wait()
        base = wid * B_PER_W
        for j in range(N_CHUNKS):
            pltpu.sync_copy(rows_v.at[j], out_hbm.at[pl.ds(base + j * CHUNK, CHUNK)])

    return gather_k(speaker_table, idx3)


def _tc_concat(sentence_embeddings, gathered):
    bm = 512
    grid = BATCH // bm

    def body(s_ref, g_ref, o_ref):
        o_ref[:, :SENT_DIM] = s_ref[...]
        o_ref[:, SENT_DIM:] = g_ref[...]

    return pl.pallas_call(
        body,
        grid=(grid,),
        in_specs=[
            pl.BlockSpec((bm, SENT_DIM), lambda i: (i, 0)),
            pl.BlockSpec((bm, SPEAKER_DIM), lambda i: (i, 0)),
        ],
        out_specs=pl.BlockSpec((bm, OUT_DIM), lambda i: (i, 0)),
        out_shape=jax.ShapeDtypeStruct((BATCH, OUT_DIM), jnp.float32),
    )(sentence_embeddings, gathered)


def kernel(sentence_embeddings, speaker_ids, speaker_table):
    idx3 = speaker_ids.astype(jnp.int32).reshape(NW, N_CHUNKS, CHUNK)
    gathered = _sc_gather(speaker_table, idx3)
    return _tc_concat(sentence_embeddings, gathered)
